# baseline (speedup 1.0000x reference)
import functools

import jax
import jax.numpy as jnp
from jax import lax
from jax.experimental import pallas as pl
from jax.experimental.pallas import tpu as pltpu
from jax.experimental.pallas import tpu_sc as plsc

HIDDEN = 2048
NUM_WORKERS = 32
CHUNK = 16
NBUF = 2


def _emb_body(idx_hbm, table_hbm, out_hbm, idx_v, *rest, bpw, n_chunks):
    bufs = rest[:NBUF]
    gsems = rest[NBUF:2 * NBUF]
    wsems = rest[2 * NBUF:3 * NBUF]
    wid = lax.axis_index("s") * 2 + lax.axis_index("c")
    base = wid * bpw
    pltpu.sync_copy(idx_hbm.at[pl.ds(base, bpw)], idx_v)

    def gather(g, b):
        return pltpu.make_async_copy(
            table_hbm.at[idx_v.at[pl.ds(g * CHUNK, CHUNK)]], bufs[b], gsems[b]
        )

    def write(g, b):
        return pltpu.make_async_copy(
            bufs[b], out_hbm.at[pl.ds(base + g * CHUNK, CHUNK)], wsems[b]
        )

    for b in range(NBUF):
        gather(b, b).start()

    def body(k, carry):
        g0 = NBUF * k
        for b in range(NBUF):
            gather(g0 + b, b).wait()
            write(g0 + b, b).start()
        for b in range(NBUF):
            g = g0 + b

            @pl.when(g + NBUF < n_chunks)
            def _():
                write(g, b).wait()
                gather(g + NBUF, b).start()

        return carry

    lax.fori_loop(0, n_chunks // NBUF, body, 0)
    for b in range(NBUF):
        write(n_chunks - NBUF + b, b).wait()


def kernel(input_ids, table):
    b, t = input_ids.shape
    n = b * t
    idx = input_ids.reshape(n).astype(jnp.int32)
    bpw = n // NUM_WORKERS
    n_chunks = bpw // CHUNK

    mesh = plsc.VectorSubcoreMesh(core_axis_name="c", subcore_axis_name="s")
    emb = pl.kernel(
        functools.partial(_emb_body, bpw=bpw, n_chunks=n_chunks),
        mesh=mesh,
        out_type=jax.ShapeDtypeStruct((n, HIDDEN), jnp.float32),
        scratch_types=(
            [pltpu.VMEM((bpw,), jnp.int32)]
            + [pltpu.VMEM((CHUNK, HIDDEN), jnp.float32)] * NBUF
            + [pltpu.SemaphoreType.DMA] * (2 * NBUF)
        ),
    )
    out = emb(idx, table)
    return out.reshape(b, t, HIDDEN)


# speedup vs baseline: 1.0549x; 1.0549x over previous
import functools

import jax
import jax.numpy as jnp
from jax import lax
from jax.experimental import pallas as pl
from jax.experimental.pallas import tpu as pltpu
from jax.experimental.pallas import tpu_sc as plsc

HIDDEN = 2048
NUM_WORKERS = 32
CHUNK = 16
NBUF = 2


def _emb_body(idx_hbm, table_hbm, out_hbm, idx_v, *rest, bpw, n_chunks):
    bufs = rest[:NBUF]
    gsems = rest[NBUF:2 * NBUF]
    wid = lax.axis_index("s") * 2 + lax.axis_index("c")
    base = wid * bpw
    pltpu.sync_copy(idx_hbm.at[pl.ds(base, bpw)], idx_v)

    def gather(g, b):
        return pltpu.make_async_copy(
            table_hbm.at[idx_v.at[pl.ds(g * CHUNK, CHUNK)]], bufs[b], gsems[b]
        )

    gather(0, 0).start()

    def body(k, carry):
        g0 = 2 * k
        gather(g0 + 1, 1).start()
        gather(g0, 0).wait()
        pltpu.sync_copy(bufs[0], out_hbm.at[pl.ds(base + g0 * CHUNK, CHUNK)])

        @pl.when(g0 + 2 < n_chunks)
        def _():
            gather(g0 + 2, 0).start()

        gather(g0 + 1, 1).wait()
        pltpu.sync_copy(
            bufs[1], out_hbm.at[pl.ds(base + (g0 + 1) * CHUNK, CHUNK)]
        )
        return carry

    lax.fori_loop(0, n_chunks // 2, body, 0)


def kernel(input_ids, table):
    b, t = input_ids.shape
    n = b * t
    idx = input_ids.reshape(n).astype(jnp.int32)
    bpw = n // NUM_WORKERS
    n_chunks = bpw // CHUNK

    mesh = plsc.VectorSubcoreMesh(core_axis_name="c", subcore_axis_name="s")
    emb = pl.kernel(
        functools.partial(_emb_body, bpw=bpw, n_chunks=n_chunks),
        mesh=mesh,
        out_type=jax.ShapeDtypeStruct((n, HIDDEN), jnp.float32),
        scratch_types=(
            [pltpu.VMEM((bpw,), jnp.int32)]
            + [pltpu.VMEM((CHUNK, HIDDEN), jnp.float32)] * NBUF
            + [pltpu.SemaphoreType.DMA] * NBUF
        ),
    )
    out = emb(idx, table)
    return out.reshape(b, t, HIDDEN)
